# Initial kernel scaffold; baseline (speedup 1.0000x reference)
import jax
import jax.numpy as jnp
from jax.experimental import pallas as pl


def kernel(x, edge_index, W, att_src, att_dst, bias):
    raise NotImplementedError("write your pallas kernel here")


# speedup vs baseline: 20.9843x; 20.9843x over previous
import functools

import jax
import jax.numpy as jnp
from jax import lax
from jax.experimental import pallas as pl
from jax.experimental.pallas import tpu as pltpu
from jax.experimental.pallas import tpu_sc as plsc

_B = 128
_ROWCHUNK = 128


def _tc_pre_body(x_ref, w_ref, asrc_w_ref, adst_w_ref,
                 h_ref, asrc_ref, adst_ref, amax_ref):
    h = jnp.dot(x_ref[...], w_ref[...], preferred_element_type=jnp.float32)
    h_ref[...] = h
    a_s = jnp.sum(h * asrc_w_ref[...], axis=1, keepdims=True)
    a_d = jnp.sum(h * adst_w_ref[...], axis=1, keepdims=True)
    asrc_ref[...] = a_s
    adst_ref[...] = a_d
    amax_ref[...] = jnp.full((1, 1), jnp.max(a_s), dtype=jnp.float32)


def _tc_merge_body(p_ref, d_ref, bias_ref, out_ref):
    n = out_ref.shape[0]
    p = p_ref[0, :n, :] + p_ref[1, :n, :]
    d = d_ref[0, :n] + d_ref[1, :n]
    out_ref[...] = p / (d[:, None] + 1e-16) + bias_ref[...][None, :]


def _sc_edge_kernel(ep, n_chunks, rows_per_tile,
                    h_hbm, asrc_hbm, adst_hbm, av_hbm, src_hbm, dst_hbm,
                    part_hbm, den_hbm,
                    asrc_v, adst_v, av_v, sidx, didx, rows, wbuf, acc, dacc,
                    sem):
    cid = lax.axis_index("c")
    sid = lax.axis_index("s")
    wid = cid * 16 + sid

    pltpu.sync_copy(asrc_hbm, asrc_v)
    pltpu.sync_copy(adst_hbm, adst_v)
    pltpu.sync_copy(av_hbm, av_v)

    zeros16 = jnp.zeros((16,), jnp.float32)

    def _zero_row(j, _):
        for k in range(8):
            rows[j, pl.ds(k * 16, 16)] = zeros16
        return 0

    lax.fori_loop(0, _ROWCHUNK, _zero_row, 0)
    for k in range(_B // 16):
        wbuf[pl.ds(k * 16, 16)] = zeros16
    row_base = sid * rows_per_tile
    for r in range(rows_per_tile // _ROWCHUNK):
        pltpu.sync_copy(rows, acc.at[pl.ds(row_base + r * _ROWCHUNK, _ROWCHUNK)])
        pltpu.sync_copy(wbuf, dacc.at[pl.ds(row_base + r * _ROWCHUNK, _B)])
    plsc.subcore_barrier()

    edges_per_tile = ep // 32

    def _chunk(g, _):
        base = wid * edges_per_tile + g * _B
        pltpu.sync_copy(src_hbm.at[pl.ds(base, _B)], sidx)
        pltpu.sync_copy(dst_hbm.at[pl.ds(base, _B)], didx)
        cp = pltpu.async_copy(h_hbm.at[sidx], rows, sem)
        av = av_v[...]
        for j in range(_B // 16):
            sv = sidx[pl.ds(j * 16, 16)]
            dv = didx[pl.ds(j * 16, 16)]
            a_s = plsc.load_gather(asrc_v, [sv])
            a_d = plsc.load_gather(adst_v, [dv])
            t = a_s + a_d
            e = jnp.where(t > 0, t, 0.2 * t)
            u = av + a_d
            m = jnp.where(u > 0, u, 0.2 * u)
            wbuf[pl.ds(j * 16, 16)] = jnp.exp(e - m)
        pltpu.sync_copy(wbuf, dacc.at[didx], add=True)
        cp.wait()

        def _scale(j, _):
            jv = jnp.full((16,), j, dtype=jnp.int32)
            ws = plsc.load_gather(wbuf, [jv])
            for k in range(8):
                rows[j, pl.ds(k * 16, 16)] = rows[j, pl.ds(k * 16, 16)] * ws
            return 0

        lax.fori_loop(0, _B, _scale, 0)
        pltpu.sync_copy(rows, acc.at[didx], add=True)
        return 0

    lax.fori_loop(0, n_chunks, _chunk, 0)
    plsc.subcore_barrier()

    for r in range(rows_per_tile // _ROWCHUNK):
        b = row_base + r * _ROWCHUNK
        pltpu.sync_copy(acc.at[pl.ds(b, _ROWCHUNK)],
                        part_hbm.at[cid, pl.ds(b, _ROWCHUNK)])
        pltpu.sync_copy(dacc.at[pl.ds(b, _ROWCHUNK)],
                        den_hbm.at[cid, pl.ds(b, _ROWCHUNK)])


def kernel(x, edge_index, W, att_src, att_dst, bias):
    n, in_ch = x.shape
    hidden = att_src.shape[1]
    e = edge_index.shape[1]

    np_ = ((n + 1 + 2047) // 2048) * 2048
    rows_per_tile = np_ // 16
    chunks_per_tile = -(-e // (32 * _B))
    ep = chunks_per_tile * 32 * _B

    x_p = jnp.pad(x, ((0, np_ - n), (0, 0)))
    src = edge_index[0].astype(jnp.int32)
    dst = edge_index[1].astype(jnp.int32)
    src_p = jnp.pad(src, (0, ep - e))
    dst_p = jnp.pad(dst, (0, ep - e), constant_values=np_ - 1)

    h, a_src2, a_dst2, amax = pl.pallas_call(
        _tc_pre_body,
        out_shape=(
            jax.ShapeDtypeStruct((np_, hidden), jnp.float32),
            jax.ShapeDtypeStruct((np_, 1), jnp.float32),
            jax.ShapeDtypeStruct((np_, 1), jnp.float32),
            jax.ShapeDtypeStruct((1, 1), jnp.float32),
        ),
    )(x_p, W, att_src[0:1, :], att_dst[0:1, :])

    a_src = a_src2.reshape(np_)
    a_dst = a_dst2.reshape(np_)
    av = jnp.broadcast_to(amax.reshape(1), (16,))

    mesh = plsc.VectorSubcoreMesh(core_axis_name="c", subcore_axis_name="s")
    sc_fn = functools.partial(_sc_edge_kernel, ep, chunks_per_tile,
                              rows_per_tile)
    part, den = pl.kernel(
        sc_fn,
        mesh=mesh,
        compiler_params=pltpu.CompilerParams(needs_layout_passes=False),
        out_type=(
            jax.ShapeDtypeStruct((2, np_, hidden), jnp.float32),
            jax.ShapeDtypeStruct((2, np_), jnp.float32),
        ),
        scratch_types=[
            pltpu.VMEM((np_,), jnp.float32),
            pltpu.VMEM((np_,), jnp.float32),
            pltpu.VMEM((16,), jnp.float32),
            pltpu.VMEM((_B,), jnp.int32),
            pltpu.VMEM((_B,), jnp.int32),
            pltpu.VMEM((_B, 128), jnp.float32),
            pltpu.VMEM((_B,), jnp.float32),
            pltpu.VMEM_SHARED((np_, 128), jnp.float32),
            pltpu.VMEM_SHARED((np_,), jnp.float32),
            pltpu.SemaphoreType.DMA,
        ],
    )(h, a_src, a_dst, av, src_p, dst_p)

    out = pl.pallas_call(
        _tc_merge_body,
        out_shape=jax.ShapeDtypeStruct((n, hidden), jnp.float32),
    )(part, den, bias)
    return out


# speedup vs baseline: 23.1059x; 1.1011x over previous
import functools

import jax
import jax.numpy as jnp
from jax import lax
from jax.experimental import pallas as pl
from jax.experimental.pallas import tpu as pltpu
from jax.experimental.pallas import tpu_sc as plsc

_B = 128
_ROWCHUNK = 128


def _tc_pre_body(x_ref, w_ref, asrc_w_ref, adst_w_ref,
                 h_ref, asrc_ref, adst_ref, amax_ref):
    h = jnp.dot(x_ref[...], w_ref[...], preferred_element_type=jnp.float32)
    h_ref[...] = h
    a_s = jnp.sum(h * asrc_w_ref[...], axis=1, keepdims=True)
    a_d = jnp.sum(h * adst_w_ref[...], axis=1, keepdims=True)
    asrc_ref[...] = a_s
    adst_ref[...] = a_d
    amax_ref[...] = jnp.full((1, 1), jnp.max(a_s), dtype=jnp.float32)


def _tc_merge_body(p_ref, d_ref, bias_ref, out_ref):
    n = out_ref.shape[0]
    p = p_ref[0, :n, :] + p_ref[1, :n, :]
    d = d_ref[0, :n] + d_ref[1, :n]
    out_ref[...] = p / (d[:, None] + 1e-16) + bias_ref[...][None, :]


def _sc_edge_kernel(n_chunks, rows_per_tile,
                    h_hbm, asrc_hbm, adst_hbm, av_hbm, eidx_hbm,
                    part_hbm, den_hbm,
                    av_v,
                    e0, e1, rows0, rows1, as0, as1, ad0, ad1, wbuf, acc, dacc,
                    semg0, semg1, sema0, sema1, semd0, semd1):
    cid = lax.axis_index("c")
    sid = lax.axis_index("s")
    wid = cid * 16 + sid
    ebufs = (e0, e1)
    rbufs = (rows0, rows1)
    asbufs = (as0, as1)
    adbufs = (ad0, ad1)
    semgs = (semg0, semg1)
    semas = (sema0, sema1)
    semds = (semd0, semd1)

    pltpu.sync_copy(av_hbm, av_v)

    zeros16 = jnp.zeros((16,), jnp.float32)

    def _zero_row(j, _):
        for k in range(8):
            rows0[j, pl.ds(k * 16, 16)] = zeros16
        return 0

    lax.fori_loop(0, _ROWCHUNK, _zero_row, 0)
    for k in range(_B // 16):
        wbuf[pl.ds(k * 16, 16)] = zeros16
    row_base = sid * rows_per_tile
    for r in range(rows_per_tile // _ROWCHUNK):
        pltpu.sync_copy(rows0, acc.at[pl.ds(row_base + r * _ROWCHUNK, _ROWCHUNK)])
        pltpu.sync_copy(wbuf, dacc.at[pl.ds(row_base + r * _ROWCHUNK, _B)])
    plsc.subcore_barrier()

    chunk0 = wid * n_chunks

    def _fetch(g, b):
        pltpu.sync_copy(eidx_hbm.at[pl.ds((chunk0 + g) * 2, 2)], ebufs[b])
        pltpu.async_copy(h_hbm.at[ebufs[b].at[0]], rbufs[b], semgs[b])
        pltpu.async_copy(asrc_hbm.at[ebufs[b].at[0]], asbufs[b], semas[b])
        pltpu.async_copy(adst_hbm.at[ebufs[b].at[1]], adbufs[b], semds[b])

    _fetch(0, 0)

    def _pair(q, _):
        for b in range(2):
            g = q * 2 + b
            eb, rb = ebufs[b], rbufs[b]

            @pl.when(g < n_chunks - 1)
            def _():
                _fetch(g + 1, 1 - b)

            av = av_v[...]
            pltpu.make_async_copy(asrc_hbm.at[eb.at[0]], asbufs[b],
                                  semas[b]).wait()
            pltpu.make_async_copy(adst_hbm.at[eb.at[1]], adbufs[b],
                                  semds[b]).wait()
            for j in range(_B // 16):
                a_s = asbufs[b][pl.ds(j * 16, 16)]
                a_d = adbufs[b][pl.ds(j * 16, 16)]
                t = a_s + a_d
                e = jnp.where(t > 0, t, 0.2 * t)
                u = av + a_d
                m = jnp.where(u > 0, u, 0.2 * u)
                wbuf[pl.ds(j * 16, 16)] = jnp.exp(e - m)
            pltpu.sync_copy(wbuf, dacc.at[eb.at[1]], add=True)
            pltpu.make_async_copy(h_hbm.at[eb.at[0]], rb, semgs[b]).wait()

            def _scale(j, _):
                jv = jnp.full((16,), j, dtype=jnp.int32)
                ws = plsc.load_gather(wbuf, [jv])
                for k in range(8):
                    rb[j, pl.ds(k * 16, 16)] = rb[j, pl.ds(k * 16, 16)] * ws
                return 0

            lax.fori_loop(0, _B, _scale, 0)
            pltpu.sync_copy(rb, acc.at[eb.at[1]], add=True)
        return 0

    lax.fori_loop(0, n_chunks // 2, _pair, 0)
    plsc.subcore_barrier()

    for r in range(rows_per_tile // _ROWCHUNK):
        bb = row_base + r * _ROWCHUNK
        pltpu.sync_copy(acc.at[pl.ds(bb, _ROWCHUNK)],
                        part_hbm.at[cid, pl.ds(bb, _ROWCHUNK)])
        pltpu.sync_copy(dacc.at[pl.ds(bb, _ROWCHUNK)],
                        den_hbm.at[cid, pl.ds(bb, _ROWCHUNK)])


def kernel(x, edge_index, W, att_src, att_dst, bias):
    n, in_ch = x.shape
    hidden = att_src.shape[1]
    e = edge_index.shape[1]

    np_ = ((n + 1 + 2047) // 2048) * 2048
    rows_per_tile = np_ // 16
    chunks_per_tile = 2 * (-(-e // (2 * 32 * _B)))
    ep = chunks_per_tile * 32 * _B

    x_p = jnp.pad(x, ((0, np_ - n), (0, 0)))
    src = edge_index[0].astype(jnp.int32)
    dst = edge_index[1].astype(jnp.int32)
    src_p = jnp.pad(src, (0, ep - e))
    dst_p = jnp.pad(dst, (0, ep - e), constant_values=np_ - 1)
    eidx = jnp.stack([src_p.reshape(-1, _B), dst_p.reshape(-1, _B)],
                     axis=1).reshape(-1, _B)

    h, a_src2, a_dst2, amax = pl.pallas_call(
        _tc_pre_body,
        out_shape=(
            jax.ShapeDtypeStruct((np_, hidden), jnp.float32),
            jax.ShapeDtypeStruct((np_, 1), jnp.float32),
            jax.ShapeDtypeStruct((np_, 1), jnp.float32),
            jax.ShapeDtypeStruct((1, 1), jnp.float32),
        ),
    )(x_p, W, att_src[0:1, :], att_dst[0:1, :])

    a_src = a_src2.reshape(np_)
    a_dst = a_dst2.reshape(np_)
    av = jnp.broadcast_to(amax.reshape(1), (16,))

    mesh = plsc.VectorSubcoreMesh(core_axis_name="c", subcore_axis_name="s")
    sc_fn = functools.partial(_sc_edge_kernel, chunks_per_tile, rows_per_tile)
    part, den = pl.kernel(
        sc_fn,
        mesh=mesh,
        compiler_params=pltpu.CompilerParams(needs_layout_passes=False),
        out_type=(
            jax.ShapeDtypeStruct((2, np_, hidden), jnp.float32),
            jax.ShapeDtypeStruct((2, np_), jnp.float32),
        ),
        scratch_types=[
            pltpu.VMEM((16,), jnp.float32),
            pltpu.VMEM((2, _B), jnp.int32),
            pltpu.VMEM((2, _B), jnp.int32),
            pltpu.VMEM((_B, 128), jnp.float32),
            pltpu.VMEM((_B, 128), jnp.float32),
            pltpu.VMEM((_B,), jnp.float32),
            pltpu.VMEM((_B,), jnp.float32),
            pltpu.VMEM((_B,), jnp.float32),
            pltpu.VMEM((_B,), jnp.float32),
            pltpu.VMEM((_B,), jnp.float32),
            pltpu.VMEM_SHARED((np_, 128), jnp.float32),
            pltpu.VMEM_SHARED((np_,), jnp.float32),
            pltpu.SemaphoreType.DMA,
            pltpu.SemaphoreType.DMA,
            pltpu.SemaphoreType.DMA,
            pltpu.SemaphoreType.DMA,
            pltpu.SemaphoreType.DMA,
            pltpu.SemaphoreType.DMA,
        ],
    )(h, a_src, a_dst, av, eidx)

    out = pl.pallas_call(
        _tc_merge_body,
        out_shape=jax.ShapeDtypeStruct((n, hidden), jnp.float32),
    )(part, den, bias)
    return out


# speedup vs baseline: 23.1908x; 1.0037x over previous
import functools

import jax
import jax.numpy as jnp
from jax import lax
from jax.experimental import pallas as pl
from jax.experimental.pallas import tpu as pltpu
from jax.experimental.pallas import tpu_sc as plsc

_B = 128
_ROWCHUNK = 128


def _tc_pre_body(x_ref, w_ref, asrc_w_ref, adst_w_ref,
                 h_ref, asrc_ref, adst_ref, amax_ref):
    h = jnp.dot(x_ref[...], w_ref[...], preferred_element_type=jnp.float32)
    h_ref[...] = h
    a_s = jnp.sum(h * asrc_w_ref[...], axis=1, keepdims=True)
    a_d = jnp.sum(h * adst_w_ref[...], axis=1, keepdims=True)
    asrc_ref[...] = a_s
    adst_ref[...] = a_d
    amax_ref[...] = jnp.full((1, 1), jnp.max(a_s), dtype=jnp.float32)


def _tc_merge_body(p_ref, d_ref, bias_ref, out_ref):
    n = out_ref.shape[0]
    p = p_ref[0, :n, :] + p_ref[1, :n, :]
    d = d_ref[0, :n] + d_ref[1, :n]
    out_ref[...] = p / (d[:, None] + 1e-16) + bias_ref[...][None, :]


def _sc_edge_kernel(n_chunks, rows_per_tile,
                    h_hbm, asrc_hbm, adst_hbm, av_hbm, eidx_hbm,
                    part_hbm, den_hbm,
                    av_v,
                    e0, e1, rows0, rows1, as0, as1, ad0, ad1, w0, w1,
                    acc, dacc,
                    semg0, semg1, sema0, sema1, semd0, semd1,
                    sems0, sems1, semw0, semw1):
    cid = lax.axis_index("c")
    sid = lax.axis_index("s")
    wid = cid * 16 + sid
    ebufs = (e0, e1)
    rbufs = (rows0, rows1)
    asbufs = (as0, as1)
    adbufs = (ad0, ad1)
    wbufs = (w0, w1)
    semgs = (semg0, semg1)
    semas = (sema0, sema1)
    semds = (semd0, semd1)
    semss = (sems0, sems1)
    semws = (semw0, semw1)

    pltpu.sync_copy(av_hbm, av_v)

    zeros16 = jnp.zeros((16,), jnp.float32)

    def _zero_row(j, _):
        for k in range(8):
            rows0[j, pl.ds(k * 16, 16)] = zeros16
        return 0

    lax.fori_loop(0, _ROWCHUNK, _zero_row, 0)
    for k in range(_B // 16):
        w0[pl.ds(k * 16, 16)] = zeros16
    row_base = sid * rows_per_tile
    for r in range(rows_per_tile // _ROWCHUNK):
        pltpu.sync_copy(rows0, acc.at[pl.ds(row_base + r * _ROWCHUNK, _ROWCHUNK)])
        pltpu.sync_copy(w0, dacc.at[pl.ds(row_base + r * _ROWCHUNK, _B)])
    plsc.subcore_barrier()

    chunk0 = wid * n_chunks

    def _fetch(g, b):
        pltpu.sync_copy(eidx_hbm.at[pl.ds((chunk0 + g) * 2, 2)], ebufs[b])
        pltpu.async_copy(h_hbm.at[ebufs[b].at[0]], rbufs[b], semgs[b])
        pltpu.async_copy(asrc_hbm.at[ebufs[b].at[0]], asbufs[b], semas[b])
        pltpu.async_copy(adst_hbm.at[ebufs[b].at[1]], adbufs[b], semds[b])

    _fetch(0, 0)

    def _pair(q, _):
        for b in range(2):
            g = q * 2 + b
            eb, rb, wb = ebufs[b], rbufs[b], wbufs[b]
            bn = 1 - b

            @pl.when(g < n_chunks - 1)
            def _():
                @pl.when(g >= 1)
                def _():
                    pltpu.make_async_copy(
                        rbufs[bn], acc.at[ebufs[bn].at[1]], semss[bn]).wait()
                _fetch(g + 1, bn)

            av = av_v[...]
            pltpu.make_async_copy(asrc_hbm.at[eb.at[0]], asbufs[b],
                                  semas[b]).wait()
            pltpu.make_async_copy(adst_hbm.at[eb.at[1]], adbufs[b],
                                  semds[b]).wait()

            @pl.when(g >= 2)
            def _():
                pltpu.make_async_copy(wb, dacc.at[eb.at[1]], semws[b]).wait()
            for j in range(_B // 16):
                a_s = asbufs[b][pl.ds(j * 16, 16)]
                a_d = adbufs[b][pl.ds(j * 16, 16)]
                t = a_s + a_d
                e = jnp.where(t > 0, t, 0.2 * t)
                u = av + a_d
                m = jnp.where(u > 0, u, 0.2 * u)
                wb[pl.ds(j * 16, 16)] = jnp.exp(e - m)
            pltpu.async_copy(wb, dacc.at[eb.at[1]], semws[b], add=True)
            pltpu.make_async_copy(h_hbm.at[eb.at[0]], rb, semgs[b]).wait()

            def _scale(j, _):
                jv = jnp.full((16,), j, dtype=jnp.int32)
                ws = plsc.load_gather(wb, [jv])
                for k in range(8):
                    rb[j, pl.ds(k * 16, 16)] = rb[j, pl.ds(k * 16, 16)] * ws
                return 0

            lax.fori_loop(0, _B, _scale, 0)
            pltpu.async_copy(rb, acc.at[eb.at[1]], semss[b], add=True)
        return 0

    lax.fori_loop(0, n_chunks // 2, _pair, 0)
    for b in range(2):
        pltpu.make_async_copy(rbufs[b], acc.at[ebufs[b].at[1]],
                              semss[b]).wait()
        pltpu.make_async_copy(wbufs[b], dacc.at[ebufs[b].at[1]],
                              semws[b]).wait()
    plsc.subcore_barrier()

    for r in range(rows_per_tile // _ROWCHUNK):
        bb = row_base + r * _ROWCHUNK
        pltpu.sync_copy(acc.at[pl.ds(bb, _ROWCHUNK)],
                        part_hbm.at[cid, pl.ds(bb, _ROWCHUNK)])
        pltpu.sync_copy(dacc.at[pl.ds(bb, _ROWCHUNK)],
                        den_hbm.at[cid, pl.ds(bb, _ROWCHUNK)])


def kernel(x, edge_index, W, att_src, att_dst, bias):
    n, in_ch = x.shape
    hidden = att_src.shape[1]
    e = edge_index.shape[1]

    np_ = ((n + 1 + 2047) // 2048) * 2048
    rows_per_tile = np_ // 16
    chunks_per_tile = 2 * (-(-e // (2 * 32 * _B)))
    ep = chunks_per_tile * 32 * _B

    x_p = jnp.pad(x, ((0, np_ - n), (0, 0)))
    src = edge_index[0].astype(jnp.int32)
    dst = edge_index[1].astype(jnp.int32)
    src_p = jnp.pad(src, (0, ep - e))
    dst_p = jnp.pad(dst, (0, ep - e), constant_values=np_ - 1)
    eidx = jnp.stack([src_p.reshape(-1, _B), dst_p.reshape(-1, _B)],
                     axis=1).reshape(-1, _B)

    h, a_src2, a_dst2, amax = pl.pallas_call(
        _tc_pre_body,
        out_shape=(
            jax.ShapeDtypeStruct((np_, hidden), jnp.float32),
            jax.ShapeDtypeStruct((np_, 1), jnp.float32),
            jax.ShapeDtypeStruct((np_, 1), jnp.float32),
            jax.ShapeDtypeStruct((1, 1), jnp.float32),
        ),
    )(x_p, W, att_src[0:1, :], att_dst[0:1, :])

    a_src = a_src2.reshape(np_)
    a_dst = a_dst2.reshape(np_)
    av = jnp.broadcast_to(amax.reshape(1), (16,))

    mesh = plsc.VectorSubcoreMesh(core_axis_name="c", subcore_axis_name="s")
    sc_fn = functools.partial(_sc_edge_kernel, chunks_per_tile, rows_per_tile)
    part, den = pl.kernel(
        sc_fn,
        mesh=mesh,
        compiler_params=pltpu.CompilerParams(needs_layout_passes=False),
        out_type=(
            jax.ShapeDtypeStruct((2, np_, hidden), jnp.float32),
            jax.ShapeDtypeStruct((2, np_), jnp.float32),
        ),
        scratch_types=[
            pltpu.VMEM((16,), jnp.float32),
            pltpu.VMEM((2, _B), jnp.int32),
            pltpu.VMEM((2, _B), jnp.int32),
            pltpu.VMEM((_B, 128), jnp.float32),
            pltpu.VMEM((_B, 128), jnp.float32),
            pltpu.VMEM((_B,), jnp.float32),
            pltpu.VMEM((_B,), jnp.float32),
            pltpu.VMEM((_B,), jnp.float32),
            pltpu.VMEM((_B,), jnp.float32),
            pltpu.VMEM((_B,), jnp.float32),
            pltpu.VMEM((_B,), jnp.float32),
            pltpu.VMEM_SHARED((np_, 128), jnp.float32),
            pltpu.VMEM_SHARED((np_,), jnp.float32),
        ] + [pltpu.SemaphoreType.DMA] * 10,
    )(h, a_src, a_dst, av, eidx)

    out = pl.pallas_call(
        _tc_merge_body,
        out_shape=jax.ShapeDtypeStruct((n, hidden), jnp.float32),
    )(part, den, bias)
    return out


# speedup vs baseline: 24.7241x; 1.0661x over previous
import functools

import jax
import jax.numpy as jnp
from jax import lax
from jax.experimental import pallas as pl
from jax.experimental.pallas import tpu as pltpu
from jax.experimental.pallas import tpu_sc as plsc

_B = 128
_ROWCHUNK = 128
_CORE0_FRAC = 0.65


def _tc_pre_body(x_ref, w_ref, asrc_w_ref, adst_w_ref,
                 h_ref, asrc_ref, adst_ref, amax_ref):
    h = jnp.dot(x_ref[...], w_ref[...], preferred_element_type=jnp.float32)
    h_ref[...] = h
    a_s = jnp.sum(h * asrc_w_ref[...], axis=1, keepdims=True)
    a_d = jnp.sum(h * adst_w_ref[...], axis=1, keepdims=True)
    asrc_ref[...] = a_s
    adst_ref[...] = a_d
    amax_ref[...] = jnp.full((1, 1), jnp.max(a_s), dtype=jnp.float32)


def _tc_merge_body(p_ref, d_ref, bias_ref, out_ref):
    n = out_ref.shape[0]
    p = p_ref[0, :n, :] + p_ref[1, :n, :]
    d = d_ref[0, :n] + d_ref[1, :n]
    out_ref[...] = p / (d[:, None] + 1e-16) + bias_ref[...][None, :]


def _sc_edge_kernel(n_chunks0, n_chunks1, rows_per_tile,
                    h_hbm, asrc_hbm, adst_hbm, av_hbm, eidx_hbm,
                    part_hbm, den_hbm,
                    av_v,
                    e0, e1, rows0, rows1, as0, as1, ad0, ad1, w0, w1,
                    acc, dacc,
                    semg0, semg1, sema0, sema1, semd0, semd1,
                    sems0, sems1, semw0, semw1):
    cid = lax.axis_index("c")
    sid = lax.axis_index("s")
    wid = cid * 16 + sid
    ebufs = (e0, e1)
    rbufs = (rows0, rows1)
    asbufs = (as0, as1)
    adbufs = (ad0, ad1)
    wbufs = (w0, w1)
    semgs = (semg0, semg1)
    semas = (sema0, sema1)
    semds = (semd0, semd1)
    semss = (sems0, sems1)
    semws = (semw0, semw1)

    pltpu.sync_copy(av_hbm, av_v)

    zeros16 = jnp.zeros((16,), jnp.float32)

    def _zero_row(j, _):
        for k in range(8):
            rows0[j, pl.ds(k * 16, 16)] = zeros16
        return 0

    lax.fori_loop(0, _ROWCHUNK, _zero_row, 0)
    for k in range(_B // 16):
        w0[pl.ds(k * 16, 16)] = zeros16
    row_base = sid * rows_per_tile
    for r in range(rows_per_tile // _ROWCHUNK):
        pltpu.sync_copy(rows0, acc.at[pl.ds(row_base + r * _ROWCHUNK, _ROWCHUNK)])
        pltpu.sync_copy(w0, dacc.at[pl.ds(row_base + r * _ROWCHUNK, _B)])
    plsc.subcore_barrier()

    is0 = cid == 0
    n_my = jnp.where(is0, n_chunks0, n_chunks1)
    chunk0 = jnp.where(is0, sid * n_chunks0, 16 * n_chunks0 + sid * n_chunks1)

    def _fetch(g, b):
        pltpu.sync_copy(eidx_hbm.at[pl.ds((chunk0 + g) * 2, 2)], ebufs[b])
        pltpu.async_copy(h_hbm.at[ebufs[b].at[0]], rbufs[b], semgs[b])
        pltpu.async_copy(asrc_hbm.at[ebufs[b].at[0]], asbufs[b], semas[b])
        pltpu.async_copy(adst_hbm.at[ebufs[b].at[1]], adbufs[b], semds[b])

    _fetch(0, 0)

    def _pair(q, _):
        for b in range(2):
            g = q * 2 + b
            eb, rb, wb = ebufs[b], rbufs[b], wbufs[b]
            bn = 1 - b

            @pl.when(g < n_my - 1)
            def _():
                @pl.when(g >= 1)
                def _():
                    pltpu.make_async_copy(
                        rbufs[bn], acc.at[ebufs[bn].at[1]], semss[bn]).wait()
                _fetch(g + 1, bn)

            av = av_v[...]
            pltpu.make_async_copy(asrc_hbm.at[eb.at[0]], asbufs[b],
                                  semas[b]).wait()
            pltpu.make_async_copy(adst_hbm.at[eb.at[1]], adbufs[b],
                                  semds[b]).wait()

            @pl.when(g >= 2)
            def _():
                pltpu.make_async_copy(wb, dacc.at[eb.at[1]], semws[b]).wait()
            for j in range(_B // 16):
                a_s = asbufs[b][pl.ds(j * 16, 16)]
                a_d = adbufs[b][pl.ds(j * 16, 16)]
                t = a_s + a_d
                e = jnp.where(t > 0, t, 0.2 * t)
                u = av + a_d
                m = jnp.where(u > 0, u, 0.2 * u)
                wb[pl.ds(j * 16, 16)] = jnp.exp(e - m)
            pltpu.async_copy(wb, dacc.at[eb.at[1]], semws[b], add=True)
            pltpu.make_async_copy(h_hbm.at[eb.at[0]], rb, semgs[b]).wait()

            def _scale(j, _):
                jv = jnp.full((16,), j, dtype=jnp.int32)
                ws = plsc.load_gather(wb, [jv])
                for k in range(8):
                    rb[j, pl.ds(k * 16, 16)] = rb[j, pl.ds(k * 16, 16)] * ws
                return 0

            lax.fori_loop(0, _B, _scale, 0)
            pltpu.async_copy(rb, acc.at[eb.at[1]], semss[b], add=True)
        return 0

    lax.fori_loop(0, n_my // 2, _pair, 0)
    for b in range(2):
        pltpu.make_async_copy(rbufs[b], acc.at[ebufs[b].at[1]],
                              semss[b]).wait()
        pltpu.make_async_copy(wbufs[b], dacc.at[ebufs[b].at[1]],
                              semws[b]).wait()
    plsc.subcore_barrier()

    for r in range(rows_per_tile // _ROWCHUNK):
        bb = row_base + r * _ROWCHUNK
        pltpu.sync_copy(acc.at[pl.ds(bb, _ROWCHUNK)],
                        part_hbm.at[cid, pl.ds(bb, _ROWCHUNK)])
        pltpu.sync_copy(dacc.at[pl.ds(bb, _ROWCHUNK)],
                        den_hbm.at[cid, pl.ds(bb, _ROWCHUNK)])


def kernel(x, edge_index, W, att_src, att_dst, bias):
    n, in_ch = x.shape
    hidden = att_src.shape[1]
    e = edge_index.shape[1]

    np_ = ((n + 1 + 2047) // 2048) * 2048
    rows_per_tile = np_ // 16
    chunks_per_tile = 2 * (-(-e // (2 * 32 * _B)))
    ep = chunks_per_tile * 32 * _B

    x_p = jnp.pad(x, ((0, np_ - n), (0, 0)))
    src = edge_index[0].astype(jnp.int32)
    dst = edge_index[1].astype(jnp.int32)
    src_p = jnp.pad(src, (0, ep - e))
    dst_p = jnp.pad(dst, (0, ep - e), constant_values=np_ - 1)
    eidx = jnp.stack([src_p.reshape(-1, _B), dst_p.reshape(-1, _B)],
                     axis=1).reshape(-1, _B)

    h, a_src2, a_dst2, amax = pl.pallas_call(
        _tc_pre_body,
        out_shape=(
            jax.ShapeDtypeStruct((np_, hidden), jnp.float32),
            jax.ShapeDtypeStruct((np_, 1), jnp.float32),
            jax.ShapeDtypeStruct((np_, 1), jnp.float32),
            jax.ShapeDtypeStruct((1, 1), jnp.float32),
        ),
    )(x_p, W, att_src[0:1, :], att_dst[0:1, :])

    a_src = a_src2.reshape(np_)
    a_dst = a_dst2.reshape(np_)
    av = jnp.broadcast_to(amax.reshape(1), (16,))

    n_pair = chunks_per_tile * 2
    n0 = 2 * int(round(_CORE0_FRAC * n_pair / 2))
    n0 = min(max(n0, 2), n_pair - 2)
    n1 = n_pair - n0

    mesh = plsc.VectorSubcoreMesh(core_axis_name="c", subcore_axis_name="s")
    sc_fn = functools.partial(_sc_edge_kernel, n0, n1, rows_per_tile)
    part, den = pl.kernel(
        sc_fn,
        mesh=mesh,
        compiler_params=pltpu.CompilerParams(needs_layout_passes=False),
        out_type=(
            jax.ShapeDtypeStruct((2, np_, hidden), jnp.float32),
            jax.ShapeDtypeStruct((2, np_), jnp.float32),
        ),
        scratch_types=[
            pltpu.VMEM((16,), jnp.float32),
            pltpu.VMEM((2, _B), jnp.int32),
            pltpu.VMEM((2, _B), jnp.int32),
            pltpu.VMEM((_B, 128), jnp.float32),
            pltpu.VMEM((_B, 128), jnp.float32),
            pltpu.VMEM((_B,), jnp.float32),
            pltpu.VMEM((_B,), jnp.float32),
            pltpu.VMEM((_B,), jnp.float32),
            pltpu.VMEM((_B,), jnp.float32),
            pltpu.VMEM((_B,), jnp.float32),
            pltpu.VMEM((_B,), jnp.float32),
            pltpu.VMEM_SHARED((np_, 128), jnp.float32),
            pltpu.VMEM_SHARED((np_,), jnp.float32),
        ] + [pltpu.SemaphoreType.DMA] * 10,
    )(h, a_src, a_dst, av, eidx)

    out = pl.pallas_call(
        _tc_merge_body,
        out_shape=jax.ShapeDtypeStruct((n, hidden), jnp.float32),
    )(part, den, bias)
    return out
